# baseline (speedup 1.0000x reference)
import functools

import jax
import jax.numpy as jnp
from jax import lax
from jax.experimental import pallas as pl
from jax.experimental.pallas import tpu as pltpu, tpu_sc as plsc


@functools.cache
def _make_gather(batch, hist, v, d):
    info = plsc.get_sparse_core_info()
    nc, ns = info.num_cores, info.num_subcores
    nw = nc * ns
    bc = batch // nw
    nct = bc // 128
    sub = bc // 2
    nds = sub // 128
    dt = d // 8
    ntask = hist * 2
    assert sub % 128 == 0 and d % 8 == 0 and ntask % 4 == 0

    mesh = plsc.VectorSubcoreMesh(core_axis_name="c", subcore_axis_name="s")

    @functools.partial(
        pl.kernel,
        mesh=mesh,
        compiler_params=pltpu.CompilerParams(
            use_tc_tiling_on_sc=False, needs_layout_passes=False),
        out_type=jax.ShapeDtypeStruct((hist, dt, batch // 128, 8, 128),
                                      jnp.float32),
        scratch_types=[
            pltpu.VMEM((hist, nct, 128), jnp.int32),
            pltpu.VMEM((sub, d), jnp.float32),
            pltpu.VMEM((sub, d), jnp.float32),
            pltpu.VMEM((sub, d), jnp.float32),
            pltpu.VMEM((sub, d), jnp.float32),
            pltpu.VMEM((d, sub), jnp.float32),
            pltpu.VMEM((d, sub), jnp.float32),
            pltpu.SemaphoreType.DMA,
            pltpu.SemaphoreType.DMA,
            pltpu.SemaphoreType.DMA,
            pltpu.SemaphoreType.DMA,
            pltpu.SemaphoreType.DMA,
            pltpu.SemaphoreType.DMA,
        ],
    )
    def gather(table_hbm, x3_hbm, out_hbm, idx_v, r0, r1, r2, r3, tr0, tr1,
               s0, s1, s2, s3, sf0, sf1):
        wid = lax.axis_index("s") * nc + lax.axis_index("c")
        rows = (r0, r1, r2, r3)
        sems = (s0, s1, s2, s3)
        trs = (tr0, tr1)
        sfs = (sf0, sf1)

        pltpu.sync_copy(x3_hbm.at[:, pl.ds(wid * nct, nct), :], idx_v)

        viota = lax.iota(jnp.int32, 16)

        def fire(t, r, sem):
            h, s = t // 2, t % 2
            for j in range(nds):
                pltpu.async_copy(
                    table_hbm.at[idx_v.at[h, s * nds + j]],
                    r.at[pl.ds(j * 128, 128)],
                    sem,
                )

        def drain(r, sem):
            pltpu.make_async_copy(table_hbm.at[pl.ds(0, sub)], r, sem).wait()

        def flush_pieces(t, tr):
            h, s = t // 2, t % 2
            for rt in range(dt):
                for ct in range(nds):
                    yield (tr.at[pl.ds(rt * 8, 8), pl.ds(ct * 128, 128)],
                           out_hbm.at[h, rt, wid * nct + s * nds + ct, :, :])

        def flush(t, tr, sem):
            for src, dst in flush_pieces(t, tr):
                pltpu.async_copy(src, dst, sem)

        def flush_wait(t, tr, sem):
            for src, dst in flush_pieces(t, tr):
                pltpu.make_async_copy(src, dst, sem).wait()

        def trans(r, tr):
            def blk_body(blk, carry):
                ridx = viota + blk * 16
                off = blk * 16

                def g(dd):
                    return plsc.load_gather(
                        r, [ridx, jnp.full((16,), dd, jnp.int32)])

                vs = [g(0), g(1), g(2), g(3)]
                for dd in range(4, d):
                    nxt = g(dd)
                    tr[dd - 4, pl.ds(off, 16)] = vs[0]
                    vs = [vs[1], vs[2], vs[3], nxt]
                for k in range(4):
                    tr[d - 4 + k, pl.ds(off, 16)] = vs[k]
                return carry

            lax.fori_loop(0, sub // 16, blk_body, 0)

        fire(0, r0, s0)
        fire(1, r1, s1)
        fire(2, r2, s2)

        def quad(q, carry):
            for i in range(4):
                t = 4 * q + i

                @pl.when(t + 3 < ntask)
                def _():
                    fire(t + 3, rows[(i + 3) % 4], sems[(i + 3) % 4])

                drain(rows[i], sems[i])

                @pl.when(t >= 2)
                def _():
                    flush_wait(t - 2, trs[i % 2], sfs[i % 2])

                trans(rows[i], trs[i % 2])
                flush(t, trs[i % 2], sfs[i % 2])
            return carry

        lax.fori_loop(0, ntask // 4, quad, 0)
        flush_wait(ntask - 2, tr0, sf0)
        flush_wait(ntask - 1, tr1, sf1)

    return gather


def kernel(x, weight):
    b, h = x.shape
    v, d = weight.shape
    x3 = x.T.reshape(h, b // 128, 128).astype(jnp.int32)
    out5 = _make_gather(b, h, v, d)(weight, x3)
    return out5.transpose(2, 4, 0, 1, 3).reshape(b, h, d)


# speedup vs baseline: 1.1660x; 1.1660x over previous
import functools

import jax
import jax.numpy as jnp
from jax import lax
from jax.experimental import pallas as pl
from jax.experimental.pallas import tpu as pltpu, tpu_sc as plsc


@functools.cache
def _make_gather(batch, hist, v, d):
    info = plsc.get_sparse_core_info()
    nc, ns = info.num_cores, info.num_subcores
    nw = nc * ns
    bc = batch // nw
    nct = bc // 128
    sub = bc // 2
    nds = sub // 128
    dt = d // 8
    ntask = hist * 2
    assert sub % 128 == 0 and d % 8 == 0 and ntask % 4 == 0

    mesh = plsc.VectorSubcoreMesh(core_axis_name="c", subcore_axis_name="s")

    @functools.partial(
        pl.kernel,
        mesh=mesh,
        compiler_params=pltpu.CompilerParams(
            use_tc_tiling_on_sc=False, needs_layout_passes=False),
        out_type=jax.ShapeDtypeStruct((hist, dt, batch // 128, 8, 128),
                                      jnp.float32),
        scratch_types=[
            pltpu.VMEM((hist, nct, 128), jnp.int32),
            pltpu.VMEM((sub, d), jnp.float32),
            pltpu.VMEM((sub, d), jnp.float32),
            pltpu.VMEM((sub, d), jnp.float32),
            pltpu.VMEM((sub, d), jnp.float32),
            pltpu.VMEM((d, sub), jnp.float32),
            pltpu.VMEM((d, sub), jnp.float32),
            pltpu.SemaphoreType.DMA,
            pltpu.SemaphoreType.DMA,
            pltpu.SemaphoreType.DMA,
            pltpu.SemaphoreType.DMA,
            pltpu.SemaphoreType.DMA,
            pltpu.SemaphoreType.DMA,
        ],
    )
    def gather(table_hbm, x3_hbm, out_hbm, idx_v, r0, r1, r2, r3, tr0, tr1,
               s0, s1, s2, s3, sf0, sf1):
        wid = lax.axis_index("s") * nc + lax.axis_index("c")
        rows = (r0, r1, r2, r3)
        sems = (s0, s1, s2, s3)
        trs = (tr0, tr1)
        sfs = (sf0, sf1)

        pltpu.sync_copy(x3_hbm.at[:, pl.ds(wid * nct, nct), :], idx_v)

        viota = lax.iota(jnp.int32, 16)

        def fire(t, r, sem):
            h, s = t // 2, t % 2
            for j in range(nds):
                pltpu.async_copy(
                    table_hbm.at[idx_v.at[h, s * nds + j]],
                    r.at[pl.ds(j * 128, 128)],
                    sem,
                )

        def drain(r, sem):
            pltpu.make_async_copy(table_hbm.at[pl.ds(0, sub)], r, sem).wait()

        def flush_pieces(t, tr):
            h, s = t // 2, t % 2
            for rt in range(dt):
                for ct in range(nds):
                    yield (tr.at[pl.ds(rt * 8, 8), pl.ds(ct * 128, 128)],
                           out_hbm.at[h, rt, wid * nct + s * nds + ct, :, :])

        def flush(t, tr, sem):
            for src, dst in flush_pieces(t, tr):
                pltpu.async_copy(src, dst, sem)

        def flush_wait(t, tr, sem):
            for src, dst in flush_pieces(t, tr):
                pltpu.make_async_copy(src, dst, sem).wait()

        def trans(r, tr):
            def blk_body(blk, carry):
                ridx = viota + blk * 16
                for dd in range(d):
                    col = jnp.bitwise_and(viota + dd, d - 1)
                    v = plsc.load_gather(r, [ridx, col])
                    plsc.store_scatter(tr, [col, ridx], v)
                return carry

            lax.fori_loop(0, sub // 16, blk_body, 0)

        fire(0, r0, s0)
        fire(1, r1, s1)
        fire(2, r2, s2)

        def quad(q, carry):
            for i in range(4):
                t = 4 * q + i

                @pl.when(t + 3 < ntask)
                def _():
                    fire(t + 3, rows[(i + 3) % 4], sems[(i + 3) % 4])

                drain(rows[i], sems[i])

                @pl.when(t >= 2)
                def _():
                    flush_wait(t - 2, trs[i % 2], sfs[i % 2])

                trans(rows[i], trs[i % 2])
                flush(t, trs[i % 2], sfs[i % 2])
            return carry

        lax.fori_loop(0, ntask // 4, quad, 0)
        flush_wait(ntask - 2, tr0, sf0)
        flush_wait(ntask - 1, tr1, sf1)

    return gather


def kernel(x, weight):
    b, h = x.shape
    v, d = weight.shape
    x3 = x.T.reshape(h, b // 128, 128).astype(jnp.int32)
    out5 = _make_gather(b, h, v, d)(weight, x3)
    return out5.transpose(2, 4, 0, 1, 3).reshape(b, h, d)


# speedup vs baseline: 1.3732x; 1.1777x over previous
import functools

import jax
import jax.numpy as jnp
from jax import lax
from jax.experimental import pallas as pl
from jax.experimental.pallas import tpu as pltpu, tpu_sc as plsc


@functools.cache
def _make_gather(batch, hist, v, d):
    info = plsc.get_sparse_core_info()
    nc, ns = info.num_cores, info.num_subcores
    nw = nc * ns
    bc = batch // nw
    nct = bc // 128
    sub = bc // 2
    nds = sub // 128
    dt = d // 8
    ntask = hist * 2
    assert sub % 128 == 0 and d % 8 == 0 and ntask % 4 == 0

    mesh = plsc.VectorSubcoreMesh(core_axis_name="c", subcore_axis_name="s")

    @functools.partial(
        pl.kernel,
        mesh=mesh,
        compiler_params=pltpu.CompilerParams(
            use_tc_tiling_on_sc=False, needs_layout_passes=False),
        out_type=jax.ShapeDtypeStruct((hist, dt, batch // 128, 8, 128),
                                      jnp.float32),
        scratch_types=[
            pltpu.VMEM((hist, nct, 128), jnp.int32),
            pltpu.VMEM((sub, d), jnp.float32),
            pltpu.VMEM((sub, d), jnp.float32),
            pltpu.VMEM((sub, d), jnp.float32),
            pltpu.VMEM((sub, d), jnp.float32),
            pltpu.VMEM((d, sub), jnp.float32),
            pltpu.VMEM((d, sub), jnp.float32),
            pltpu.SemaphoreType.DMA,
            pltpu.SemaphoreType.DMA,
            pltpu.SemaphoreType.DMA,
            pltpu.SemaphoreType.DMA,
            pltpu.SemaphoreType.DMA,
            pltpu.SemaphoreType.DMA,
        ],
    )
    def gather(table_hbm, x3_hbm, out_hbm, idx_v, r0, r1, r2, r3, tr0, tr1,
               s0, s1, s2, s3, sf0, sf1):
        wid = lax.axis_index("s") * nc + lax.axis_index("c")
        rows = (r0, r1, r2, r3)
        sems = (s0, s1, s2, s3)
        trs = (tr0, tr1)
        sfs = (sf0, sf1)

        pltpu.sync_copy(x3_hbm.at[:, pl.ds(wid * nct, nct), :], idx_v)

        viota = lax.iota(jnp.int32, 16)

        def fire(t, r, sem):
            h, s = t // 2, t % 2
            for j in range(nds):
                pltpu.async_copy(
                    table_hbm.at[idx_v.at[h, s * nds + j]],
                    r.at[pl.ds(j * 128, 128)],
                    sem,
                )

        def drain(r, sem):
            pltpu.make_async_copy(table_hbm.at[pl.ds(0, sub)], r, sem).wait()

        def flush_pieces(t, tr):
            h, s = t // 2, t % 2
            for rt in range(dt):
                for ct in range(nds):
                    yield (tr.at[pl.ds(rt * 8, 8), pl.ds(ct * 128, 128)],
                           out_hbm.at[h, rt, wid * nct + s * nds + ct, :, :])

        def flush(t, tr, sem):
            for src, dst in flush_pieces(t, tr):
                pltpu.async_copy(src, dst, sem)

        def flush_wait(t, tr, sem):
            for src, dst in flush_pieces(t, tr):
                pltpu.make_async_copy(src, dst, sem).wait()

        def trans(r, tr):
            def blk_body(blk, carry):
                ridx = viota + blk * 16

                def g(dd):
                    col = jnp.bitwise_and(viota + dd, d - 1)
                    return col, plsc.load_gather(r, [ridx, col])

                c0, v0 = g(0)
                c1, v1 = g(1)
                for dd in range(2, d):
                    c2, v2 = g(dd)
                    plsc.store_scatter(tr, [c0, ridx], v0)
                    c0, v0, c1, v1 = c1, v1, c2, v2
                plsc.store_scatter(tr, [c0, ridx], v0)
                plsc.store_scatter(tr, [c1, ridx], v1)
                return carry

            lax.fori_loop(0, sub // 16, blk_body, 0)

        fire(0, r0, s0)
        fire(1, r1, s1)
        fire(2, r2, s2)

        def quad(q, carry):
            for i in range(4):
                t = 4 * q + i

                @pl.when(t + 3 < ntask)
                def _():
                    fire(t + 3, rows[(i + 3) % 4], sems[(i + 3) % 4])

                drain(rows[i], sems[i])

                @pl.when(t >= 2)
                def _():
                    flush_wait(t - 2, trs[i % 2], sfs[i % 2])

                trans(rows[i], trs[i % 2])
                flush(t, trs[i % 2], sfs[i % 2])
            return carry

        lax.fori_loop(0, ntask // 4, quad, 0)
        flush_wait(ntask - 2, tr0, sf0)
        flush_wait(ntask - 1, tr1, sf1)

    return gather


def kernel(x, weight):
    b, h = x.shape
    v, d = weight.shape
    x3 = x.T.reshape(h, b // 128, 128).astype(jnp.int32)
    out5 = _make_gather(b, h, v, d)(weight, x3)
    return out5.transpose(2, 4, 0, 1, 3).reshape(b, h, d)


# speedup vs baseline: 1.3950x; 1.0159x over previous
import functools

import jax
import jax.numpy as jnp
from jax import lax
from jax.experimental import pallas as pl
from jax.experimental.pallas import tpu as pltpu, tpu_sc as plsc


@functools.cache
def _make_gather(batch, hist, v, d):
    info = plsc.get_sparse_core_info()
    nc, ns = info.num_cores, info.num_subcores
    nw = nc * ns
    bc = batch // nw
    nct = bc // 128
    sub = bc // 2
    nds = sub // 128
    dt = d // 8
    ntask = hist * 2
    assert sub % 128 == 0 and d % 8 == 0 and ntask % 4 == 0

    mesh = plsc.VectorSubcoreMesh(core_axis_name="c", subcore_axis_name="s")

    @functools.partial(
        pl.kernel,
        mesh=mesh,
        compiler_params=pltpu.CompilerParams(
            use_tc_tiling_on_sc=False, needs_layout_passes=False),
        out_type=jax.ShapeDtypeStruct((hist, dt, batch // 128, 8, 128),
                                      jnp.float32),
        scratch_types=[
            pltpu.VMEM((hist, nct, 128), jnp.int32),
            pltpu.VMEM((sub, d), jnp.float32),
            pltpu.VMEM((sub, d), jnp.float32),
            pltpu.VMEM((sub, d), jnp.float32),
            pltpu.VMEM((sub, d), jnp.float32),
            pltpu.VMEM((d, sub), jnp.float32),
            pltpu.VMEM((d, sub), jnp.float32),
            pltpu.SemaphoreType.DMA,
            pltpu.SemaphoreType.DMA,
            pltpu.SemaphoreType.DMA,
            pltpu.SemaphoreType.DMA,
            pltpu.SemaphoreType.DMA,
            pltpu.SemaphoreType.DMA,
        ],
    )
    def gather(table_hbm, x3_hbm, out_hbm, idx_v, r0, r1, r2, r3, tr0, tr1,
               s0, s1, s2, s3, sf0, sf1):
        wid = lax.axis_index("s") * nc + lax.axis_index("c")
        rows = (r0, r1, r2, r3)
        sems = (s0, s1, s2, s3)
        trs = (tr0, tr1)
        sfs = (sf0, sf1)

        pltpu.sync_copy(x3_hbm.at[:, pl.ds(wid * nct, nct), :], idx_v)

        viota = lax.iota(jnp.int32, 16)

        def fire(t, r, sem):
            h, s = t // 2, t % 2
            for j in range(nds):
                pltpu.async_copy(
                    table_hbm.at[idx_v.at[h, s * nds + j]],
                    r.at[pl.ds(j * 128, 128)],
                    sem,
                )

        def drain(r, sem):
            pltpu.make_async_copy(table_hbm.at[pl.ds(0, sub)], r, sem).wait()

        def flush_pieces(t, tr):
            h, s = t // 2, t % 2
            for rt in range(dt):
                for ct in range(nds):
                    yield (tr.at[pl.ds(rt * 8, 8), pl.ds(ct * 128, 128)],
                           out_hbm.at[h, rt, wid * nct + s * nds + ct, :, :])

        def flush(t, tr, sem):
            for src, dst in flush_pieces(t, tr):
                pltpu.async_copy(src, dst, sem)

        def flush_wait(t, tr, sem):
            for src, dst in flush_pieces(t, tr):
                pltpu.make_async_copy(src, dst, sem).wait()

        def trans(r, tr):
            def blk_body(blk, carry):
                ridx = viota + blk * 16

                def g(dd):
                    col = jnp.bitwise_and(viota + dd, d - 1)
                    return col, plsc.load_gather(r, [ridx, col])

                pend = [g(0), g(1), g(2)]
                for dd in range(3, d):
                    nxt = g(dd)
                    c0, v0 = pend[0]
                    plsc.store_scatter(tr, [c0, ridx], v0)
                    pend = [pend[1], pend[2], nxt]
                for c0, v0 in pend:
                    plsc.store_scatter(tr, [c0, ridx], v0)
                return carry

            lax.fori_loop(0, sub // 16, blk_body, 0)

        fire(0, r0, s0)
        fire(1, r1, s1)
        fire(2, r2, s2)

        def quad(q, carry):
            for i in range(4):
                t = 4 * q + i

                @pl.when(t + 3 < ntask)
                def _():
                    fire(t + 3, rows[(i + 3) % 4], sems[(i + 3) % 4])

                drain(rows[i], sems[i])

                @pl.when(t >= 2)
                def _():
                    flush_wait(t - 2, trs[i % 2], sfs[i % 2])

                trans(rows[i], trs[i % 2])
                flush(t, trs[i % 2], sfs[i % 2])
            return carry

        lax.fori_loop(0, ntask // 4, quad, 0)
        flush_wait(ntask - 2, tr0, sf0)
        flush_wait(ntask - 1, tr1, sf1)

    return gather


def kernel(x, weight):
    b, h = x.shape
    v, d = weight.shape
    x3 = x.T.reshape(h, b // 128, 128).astype(jnp.int32)
    out5 = _make_gather(b, h, v, d)(weight, x3)
    return out5.transpose(2, 4, 0, 1, 3).reshape(b, h, d)
